# baseline (speedup 1.0000x reference)
import functools

import jax
import jax.numpy as jnp
from jax import lax
from jax.experimental import pallas as pl
from jax.experimental.pallas import tpu as pltpu
from jax.experimental.pallas import tpu_sc as plsc

NC = 2
NS = 16
L = 16
NW = NC * NS
BATCH = 16384
F = 32
N_U = 1000000
N_M = 100000
BPW = BATCH // NW
NG = 4
GR = BPW // NG
GCH = GR // L
GF = GR * F


def _mf_body(user_hbm, movie_hbm, uf_hbm, mf_hbm, out_hbm,
             uidx_v, midx_v,
             uix0, uix1, mix0, mix1,
             ud0, ud1, md0, md1, out_v,
             su0, su1, sm0, sm1):
    wid = lax.axis_index("s") * NC + lax.axis_index("c")
    base = wid * BPW

    pltpu.sync_copy(user_hbm.at[pl.ds(base, BPW)], uidx_v)
    pltpu.sync_copy(movie_hbm.at[pl.ds(base, BPW)], midx_v)

    uix = (uix0, uix1)
    mix = (mix0, mix1)
    ud = (ud0, ud1)
    md = (md0, md1)
    usem = (su0, su1)
    msem = (sm0, sm1)

    def build(g):
        b = g % 2

        def chunk(c, carry):
            sl = pl.ds(g * GR + c * L, L)
            uc = uidx_v[sl]
            mc = midx_v[sl]
            for f in range(F):
                dsl = pl.ds(f * GR + c * L, L)
                uix[b][dsl] = uc + (f * N_U)
                mix[b][dsl] = mc + (f * N_M)
            return carry

        lax.fori_loop(0, GCH, chunk, 0)

    def fire(g):
        b = g % 2
        cu = pltpu.async_copy(uf_hbm.at[uix[b]], ud[b], usem[b])
        cm = pltpu.async_copy(mf_hbm.at[mix[b]], md[b], msem[b])
        return cu, cm

    build(0)
    pending = fire(0)

    for g in range(NG):
        if g + 1 < NG:
            build(g + 1)
        nxt = fire(g + 1) if g + 1 < NG else None
        pending[0].wait()
        pending[1].wait()
        b = g % 2
        udb, mdb = ud[b], md[b]

        def comp(c, carry, udb=udb, mdb=mdb, g=g):
            acc = jnp.zeros((L,), jnp.float32)
            for f in range(F):
                dsl = pl.ds(f * GR + c * L, L)
                acc = acc + udb[dsl] * mdb[dsl]
            out_v[pl.ds(g * GR + c * L, L)] = acc
            return carry

        lax.fori_loop(0, GCH, comp, 0)
        pending = nxt

    pltpu.sync_copy(out_v, out_hbm.at[pl.ds(base, BPW)])


@jax.jit
def kernel(user, movie, user_factors, movie_factors):
    uf1 = user_factors.T.reshape(-1)
    mf1 = movie_factors.T.reshape(-1)
    mesh = plsc.VectorSubcoreMesh(
        core_axis_name="c", subcore_axis_name="s",
        num_cores=NC, num_subcores=NS)
    run = pl.kernel(
        _mf_body,
        out_type=jax.ShapeDtypeStruct((BATCH,), jnp.float32),
        mesh=mesh,
        scratch_types=[
            pltpu.VMEM((BPW,), jnp.int32),
            pltpu.VMEM((BPW,), jnp.int32),
            pltpu.VMEM((GF,), jnp.int32),
            pltpu.VMEM((GF,), jnp.int32),
            pltpu.VMEM((GF,), jnp.int32),
            pltpu.VMEM((GF,), jnp.int32),
            pltpu.VMEM((GF,), jnp.float32),
            pltpu.VMEM((GF,), jnp.float32),
            pltpu.VMEM((GF,), jnp.float32),
            pltpu.VMEM((GF,), jnp.float32),
            pltpu.VMEM((BPW,), jnp.float32),
            pltpu.SemaphoreType.DMA,
            pltpu.SemaphoreType.DMA,
            pltpu.SemaphoreType.DMA,
            pltpu.SemaphoreType.DMA,
        ],
        compiler_params=pltpu.CompilerParams(needs_layout_passes=False),
    )
    return run(user, movie, uf1, mf1)


# speedup vs baseline: 12.9833x; 12.9833x over previous
import functools

import jax
import jax.numpy as jnp
from jax import lax
from jax.experimental import pallas as pl
from jax.experimental.pallas import tpu as pltpu
from jax.experimental.pallas import tpu_sc as plsc

NC = 2
NS = 16
L = 16
NW = NC * NS
BATCH = 16384
F = 32
N_U = 1000000
N_M = 100000
MPACK = 128 // F
BPW = BATCH // NW
NG = 4
GR = BPW // NG
GCH = GR // L
RING = 8
NSTEP = BPW // RING


def _mf_body(user_hbm, movie_hbm, ufv_hbm, mf4_hbm, out_hbm,
             uidx_v, midx_v, mridx_v,
             ut0, ut1, mr0, mr1, urows_v, out_v,
             su0, su1, sm0, sm1):
    wid = lax.axis_index("s") * NC + lax.axis_index("c")
    base = wid * BPW

    pltpu.sync_copy(user_hbm.at[pl.ds(base, BPW)], uidx_v.at[pl.ds(0, BPW)])
    pltpu.sync_copy(movie_hbm.at[pl.ds(base, BPW)], midx_v)

    def mk_mridx(i, carry):
        sl = pl.ds(i * L, L)
        mridx_v[sl] = lax.shift_right_logical(midx_v[sl], 2)
        return carry

    lax.fori_loop(0, BPW // L, mk_mridx, 0)

    mbufs = (mr0, mr1)
    msems = (sm0, sm1)

    def fire_movie(g):
        b = g % 2
        return pltpu.async_copy(
            mf4_hbm.at[mridx_v.at[pl.ds(g * GR, GR)]], mbufs[b], msems[b])

    movie_descs = [fire_movie(0)]

    iota = lax.iota(jnp.int32, L)
    f16 = iota
    band_lo = lax.shift_right_logical(f16, 3)
    band_hi = band_lo + 2
    fr16 = f16 & 7

    def fire_user(s, buf, sem):
        v = uidx_v[pl.ds(s * RING, L)]
        for j in range(RING):
            i = v[j]
            w = lax.shift_right_logical(i, 7)
            pltpu.async_copy(
                ufv_hbm.at[:, :, pl.ds(w * 128, 128)], buf.at[j], sem)

    def drain_user(buf, sem):
        for j in range(RING):
            pltpu.make_async_copy(
                ufv_hbm.at[:, :, pl.ds(0, 128)], buf.at[j], sem).wait()

    def process_user(k, buf):
        v = uidx_v[pl.ds(k * RING, L)]
        for j in range(RING):
            i = v[j]
            il = jnp.broadcast_to(i & 127, (L,))
            jb = jnp.full((L,), j, jnp.int32)
            u_lo = plsc.load_gather(buf, [jb, band_lo, fr16, il])
            u_hi = plsc.load_gather(buf, [jb, band_hi, fr16, il])
            r = k * RING + j
            urows_v[pl.ds(r * F, L)] = u_lo
            urows_v[pl.ds(r * F + L, L)] = u_hi

    fire_user(0, ut0, su0)

    def user_step(k, carry):
        @pl.when(k % 2 == 0)
        def _():
            @pl.when(k + 1 < NSTEP)
            def _():
                fire_user(k + 1, ut1, su1)
            drain_user(ut0, su0)
            process_user(k, ut0)

        @pl.when(k % 2 == 1)
        def _():
            @pl.when(k + 1 < NSTEP)
            def _():
                fire_user(k + 1, ut0, su0)
            drain_user(ut1, su1)
            process_user(k, ut1)

        return carry

    lax.fori_loop(0, NSTEP, user_step, 0)

    for g in range(NG):
        if g + 1 < NG:
            movie_descs.append(fire_movie(g + 1))
        movie_descs[g].wait()
        mrb = mbufs[g % 2]

        def comp(c, carry, mrb=mrb, g=g):
            sl = pl.ds(g * GR + c * L, L)
            moff = (midx_v[sl] & (MPACK - 1)) * F
            rows = c * L + iota
            ubase = (g * GR + c * L + iota) * F
            acc = jnp.zeros((L,), jnp.float32)
            for f in range(F):
                uv = plsc.load_gather(urows_v, [ubase + f])
                mv = plsc.load_gather(mrb, [rows, moff + f])
                acc = acc + uv * mv
            out_v[sl] = acc
            return carry

        lax.fori_loop(0, GCH, comp, 0)

    pltpu.sync_copy(out_v, out_hbm.at[pl.ds(base, BPW)])


@jax.jit
def kernel(user, movie, user_factors, movie_factors):
    n_users, n_factors = user_factors.shape
    n_movies, _ = movie_factors.shape
    ufv = user_factors.T.reshape(4, 8, n_users)
    mf4 = movie_factors.reshape(n_movies // MPACK, MPACK * n_factors)
    mesh = plsc.VectorSubcoreMesh(
        core_axis_name="c", subcore_axis_name="s",
        num_cores=NC, num_subcores=NS)
    run = pl.kernel(
        _mf_body,
        out_type=jax.ShapeDtypeStruct((BATCH,), jnp.float32),
        mesh=mesh,
        scratch_types=[
            pltpu.VMEM((BPW + L,), jnp.int32),
            pltpu.VMEM((BPW,), jnp.int32),
            pltpu.VMEM((BPW,), jnp.int32),
            pltpu.VMEM((RING, 4, 8, 128), jnp.float32),
            pltpu.VMEM((RING, 4, 8, 128), jnp.float32),
            pltpu.VMEM((GR, MPACK * F), jnp.float32),
            pltpu.VMEM((GR, MPACK * F), jnp.float32),
            pltpu.VMEM((BPW * F,), jnp.float32),
            pltpu.VMEM((BPW,), jnp.float32),
            pltpu.SemaphoreType.DMA,
            pltpu.SemaphoreType.DMA,
            pltpu.SemaphoreType.DMA,
            pltpu.SemaphoreType.DMA,
        ],
        compiler_params=pltpu.CompilerParams(needs_layout_passes=False),
    )
    return run(user, movie, ufv, mf4)


# speedup vs baseline: 13.6386x; 1.0505x over previous
import functools

import jax
import jax.numpy as jnp
from jax import lax
from jax.experimental import pallas as pl
from jax.experimental.pallas import tpu as pltpu
from jax.experimental.pallas import tpu_sc as plsc

NC = 2
NS = 16
L = 16
NW = NC * NS
BATCH = 16384
F = 32
N_U = 1000000
N_M = 100000
MPACK = 128 // F
BPW = BATCH // NW
NG = 16
GR = BPW // NG
GCH = GR // L
RING = 8
NSTEP = BPW // RING


def _mf_body(user_hbm, movie_hbm, ufv_hbm, mf4_hbm, out_hbm,
             uidx_v, midx_v, mridx_v,
             ut0, ut1, ut2, mr0, mr1, urows_v, out_v,
             su0, su1, su2, sm0, sm1):
    wid = lax.axis_index("s") * NC + lax.axis_index("c")
    base = wid * BPW

    pltpu.sync_copy(user_hbm.at[pl.ds(base, BPW)], uidx_v.at[pl.ds(0, BPW)])
    pltpu.sync_copy(movie_hbm.at[pl.ds(base, BPW)], midx_v)

    def mk_mridx(i, carry):
        sl = pl.ds(i * L, L)
        mridx_v[sl] = lax.shift_right_logical(midx_v[sl], 2)
        return carry

    lax.fori_loop(0, BPW // L, mk_mridx, 0)

    mbufs = (mr0, mr1)
    msems = (sm0, sm1)

    def fire_movie(g):
        b = g % 2
        return pltpu.async_copy(
            mf4_hbm.at[mridx_v.at[pl.ds(g * GR, GR)]], mbufs[b], msems[b])

    movie_descs = [fire_movie(0)]

    iota = lax.iota(jnp.int32, L)
    f16 = iota
    band_lo = lax.shift_right_logical(f16, 3)
    band_hi = band_lo + 2
    fr16 = f16 & 7

    def fire_user(s, buf, sem):
        v = uidx_v[pl.ds(s * RING, L)]
        for j in range(RING):
            i = v[j]
            w = lax.shift_right_logical(i, 7)
            pltpu.async_copy(
                ufv_hbm.at[:, :, pl.ds(w * 128, 128)], buf.at[j], sem)

    def drain_user(buf, sem):
        for j in range(RING):
            pltpu.make_async_copy(
                ufv_hbm.at[:, :, pl.ds(0, 128)], buf.at[j], sem).wait()

    def process_user(k, buf):
        v = uidx_v[pl.ds(k * RING, L)]
        for j in range(RING):
            i = v[j]
            il = jnp.broadcast_to(i & 127, (L,))
            jb = jnp.full((L,), j, jnp.int32)
            u_lo = plsc.load_gather(buf, [jb, band_lo, fr16, il])
            u_hi = plsc.load_gather(buf, [jb, band_hi, fr16, il])
            r = k * RING + j
            urows_v[pl.ds(r * F, L)] = u_lo
            urows_v[pl.ds(r * F + L, L)] = u_hi

    ubufs = (ut0, ut1, ut2)
    usems = (su0, su1, su2)
    fire_user(0, ut0, su0)
    fire_user(1, ut1, su1)

    def user_step(k, carry):
        for ph in range(3):
            @pl.when(k % 3 == ph)
            def _(ph=ph):
                nxt = (ph + 2) % 3

                @pl.when(k + 2 < NSTEP)
                def _():
                    fire_user(k + 2, ubufs[nxt], usems[nxt])
                drain_user(ubufs[ph], usems[ph])
                process_user(k, ubufs[ph])

        return carry

    lax.fori_loop(0, NSTEP, user_step, 0)

    for g in range(NG):
        if g + 1 < NG:
            movie_descs.append(fire_movie(g + 1))
        movie_descs[g].wait()
        mrb = mbufs[g % 2]

        def comp(c, carry, mrb=mrb, g=g):
            sl = pl.ds(g * GR + c * L, L)
            moff = (midx_v[sl] & (MPACK - 1)) * F
            rows = c * L + iota
            ubase = (g * GR + c * L + iota) * F
            acc = jnp.zeros((L,), jnp.float32)
            for f in range(F):
                uv = plsc.load_gather(urows_v, [ubase + f])
                mv = plsc.load_gather(mrb, [rows, moff + f])
                acc = acc + uv * mv
            out_v[sl] = acc
            return carry

        lax.fori_loop(0, GCH, comp, 0)

    pltpu.sync_copy(out_v, out_hbm.at[pl.ds(base, BPW)])


@jax.jit
def kernel(user, movie, user_factors, movie_factors):
    n_users, n_factors = user_factors.shape
    n_movies, _ = movie_factors.shape
    ufv = user_factors.T.reshape(4, 8, n_users)
    mf4 = movie_factors.reshape(n_movies // MPACK, MPACK * n_factors)
    mesh = plsc.VectorSubcoreMesh(
        core_axis_name="c", subcore_axis_name="s",
        num_cores=NC, num_subcores=NS)
    run = pl.kernel(
        _mf_body,
        out_type=jax.ShapeDtypeStruct((BATCH,), jnp.float32),
        mesh=mesh,
        scratch_types=[
            pltpu.VMEM((BPW + L,), jnp.int32),
            pltpu.VMEM((BPW,), jnp.int32),
            pltpu.VMEM((BPW,), jnp.int32),
            pltpu.VMEM((RING, 4, 8, 128), jnp.float32),
            pltpu.VMEM((RING, 4, 8, 128), jnp.float32),
            pltpu.VMEM((RING, 4, 8, 128), jnp.float32),
            pltpu.VMEM((GR, MPACK * F), jnp.float32),
            pltpu.VMEM((GR, MPACK * F), jnp.float32),
            pltpu.VMEM((BPW * F,), jnp.float32),
            pltpu.VMEM((BPW,), jnp.float32),
            pltpu.SemaphoreType.DMA,
            pltpu.SemaphoreType.DMA,
            pltpu.SemaphoreType.DMA,
            pltpu.SemaphoreType.DMA,
            pltpu.SemaphoreType.DMA,
        ],
        compiler_params=pltpu.CompilerParams(needs_layout_passes=False),
    )
    return run(user, movie, ufv, mf4)


# speedup vs baseline: 13.6802x; 1.0031x over previous
import functools

import jax
import jax.numpy as jnp
from jax import lax
from jax.experimental import pallas as pl
from jax.experimental.pallas import tpu as pltpu
from jax.experimental.pallas import tpu_sc as plsc

NC = 2
NS = 16
L = 16
NW = NC * NS
BATCH = 16384
F = 32
N_U = 1000000
N_M = 100000
MPACK = 128 // F
BPW = BATCH // NW
NG = 16
GR = BPW // NG
GCH = GR // L
RING = 8
NSTEP = BPW // RING


def _mf_body(user_hbm, movie_hbm, ufv_hbm, mf4_hbm, out_hbm,
             uidx_v, midx_v, mridx_v,
             ut0, ut1, ut2, mr0, mr1, urows_v, out_v,
             su0, su1, su2, sm0, sm1):
    wid = lax.axis_index("s") * NC + lax.axis_index("c")
    base = wid * BPW

    pltpu.sync_copy(user_hbm.at[pl.ds(base, BPW)], uidx_v.at[pl.ds(0, BPW)])
    pltpu.sync_copy(movie_hbm.at[pl.ds(base, BPW)], midx_v)

    def mk_mridx(i, carry):
        sl = pl.ds(i * L, L)
        mridx_v[sl] = lax.shift_right_logical(midx_v[sl], 2)
        return carry

    lax.fori_loop(0, BPW // L, mk_mridx, 0)

    mbufs = (mr0, mr1)
    msems = (sm0, sm1)

    def fire_movie(g):
        b = g % 2
        return pltpu.async_copy(
            mf4_hbm.at[mridx_v.at[pl.ds(g * GR, GR)]], mbufs[b], msems[b])

    movie_descs = [fire_movie(0)]

    iota = lax.iota(jnp.int32, L)
    f16 = iota
    band_lo = lax.shift_right_logical(f16, 3)
    band_hi = band_lo + 2
    fr16 = f16 & 7

    def fire_user(s, buf, sem):
        v = uidx_v[pl.ds(s * RING, L)]
        for j in range(RING):
            i = v[j]
            w = lax.shift_right_logical(i, 7)
            for fb in range(4):
                pltpu.async_copy(
                    ufv_hbm.at[fb, :, pl.ds(w * 128, 128)], buf.at[j, fb], sem)

    def drain_user(buf, sem):
        for j in range(RING):
            pltpu.make_async_copy(
                ufv_hbm.at[:, :, pl.ds(0, 128)], buf.at[j], sem).wait()

    def process_user(k, buf):
        v = uidx_v[pl.ds(k * RING, L)]
        for j in range(RING):
            i = v[j]
            il = jnp.broadcast_to(i & 127, (L,))
            jb = jnp.full((L,), j, jnp.int32)
            u_lo = plsc.load_gather(buf, [jb, band_lo, fr16, il])
            u_hi = plsc.load_gather(buf, [jb, band_hi, fr16, il])
            r = k * RING + j
            urows_v[pl.ds(r * F, L)] = u_lo
            urows_v[pl.ds(r * F + L, L)] = u_hi

    ubufs = (ut0, ut1, ut2)
    usems = (su0, su1, su2)
    fire_user(0, ut0, su0)
    fire_user(1, ut1, su1)

    def user_step(k, carry):
        for ph in range(3):
            @pl.when(k % 3 == ph)
            def _(ph=ph):
                nxt = (ph + 2) % 3

                @pl.when(k + 2 < NSTEP)
                def _():
                    fire_user(k + 2, ubufs[nxt], usems[nxt])
                drain_user(ubufs[ph], usems[ph])
                process_user(k, ubufs[ph])

        return carry

    lax.fori_loop(0, NSTEP, user_step, 0)

    for g in range(NG):
        if g + 1 < NG:
            movie_descs.append(fire_movie(g + 1))
        movie_descs[g].wait()
        mrb = mbufs[g % 2]

        def comp(c, carry, mrb=mrb, g=g):
            sl = pl.ds(g * GR + c * L, L)
            moff = (midx_v[sl] & (MPACK - 1)) * F
            rows = c * L + iota
            ubase = (g * GR + c * L + iota) * F
            acc = jnp.zeros((L,), jnp.float32)
            for f in range(F):
                uv = plsc.load_gather(urows_v, [ubase + f])
                mv = plsc.load_gather(mrb, [rows, moff + f])
                acc = acc + uv * mv
            out_v[sl] = acc
            return carry

        lax.fori_loop(0, GCH, comp, 0)

    pltpu.sync_copy(out_v, out_hbm.at[pl.ds(base, BPW)])


@jax.jit
def kernel(user, movie, user_factors, movie_factors):
    n_users, n_factors = user_factors.shape
    n_movies, _ = movie_factors.shape
    ufv = user_factors.T.reshape(4, 8, n_users)
    mf4 = movie_factors.reshape(n_movies // MPACK, MPACK * n_factors)
    mesh = plsc.VectorSubcoreMesh(
        core_axis_name="c", subcore_axis_name="s",
        num_cores=NC, num_subcores=NS)
    run = pl.kernel(
        _mf_body,
        out_type=jax.ShapeDtypeStruct((BATCH,), jnp.float32),
        mesh=mesh,
        scratch_types=[
            pltpu.VMEM((BPW + L,), jnp.int32),
            pltpu.VMEM((BPW,), jnp.int32),
            pltpu.VMEM((BPW,), jnp.int32),
            pltpu.VMEM((RING, 4, 8, 128), jnp.float32),
            pltpu.VMEM((RING, 4, 8, 128), jnp.float32),
            pltpu.VMEM((RING, 4, 8, 128), jnp.float32),
            pltpu.VMEM((GR, MPACK * F), jnp.float32),
            pltpu.VMEM((GR, MPACK * F), jnp.float32),
            pltpu.VMEM((BPW * F,), jnp.float32),
            pltpu.VMEM((BPW,), jnp.float32),
            pltpu.SemaphoreType.DMA,
            pltpu.SemaphoreType.DMA,
            pltpu.SemaphoreType.DMA,
            pltpu.SemaphoreType.DMA,
            pltpu.SemaphoreType.DMA,
        ],
        compiler_params=pltpu.CompilerParams(needs_layout_passes=False),
    )
    return run(user, movie, ufv, mf4)


# speedup vs baseline: 13.7503x; 1.0051x over previous
import jax
import jax.numpy as jnp
from jax import lax
from jax.experimental import pallas as pl
from jax.experimental.pallas import tpu as pltpu
from jax.experimental.pallas import tpu_sc as plsc

NC = 2
NS = 16
L = 16
NW = NC * NS
BATCH = 16384
F = 32
N_U = 1000000
N_M = 100000
MPACK = 128 // F
BPW = BATCH // NW
NG = 16
GR = BPW // NG
GCH = GR // L
RING = 8
NSTEP = BPW // RING


def _mf_body(user_hbm, movie_hbm, ufv_hbm, mf4_hbm, out_hbm,
             uidx_v, midx_v, mridx_v,
             ut0, ut1, ut2, mr0, mr1, urows_v, out_v,
             su0, su1, su2, sm0, sm1):
    wid = lax.axis_index("s") * NC + lax.axis_index("c")
    base = wid * BPW

    pltpu.sync_copy(user_hbm.at[pl.ds(base, BPW)], uidx_v.at[pl.ds(0, BPW)])
    pltpu.sync_copy(movie_hbm.at[pl.ds(base, BPW)], midx_v)

    def mk_mridx(i, carry):
        sl = pl.ds(i * L, L)
        mridx_v[sl] = lax.shift_right_logical(midx_v[sl], 2)
        return carry

    lax.fori_loop(0, BPW // L, mk_mridx, 0)

    mbufs = (mr0, mr1)
    msems = (sm0, sm1)

    def fire_movie(g):
        b = g % 2
        return pltpu.async_copy(
            mf4_hbm.at[mridx_v.at[pl.ds(g * GR, GR)]], mbufs[b], msems[b])

    movie_descs = [fire_movie(0)]

    iota = lax.iota(jnp.int32, L)
    f16 = iota
    band_lo = lax.shift_right_logical(f16, 3)
    band_hi = band_lo + 2
    fr16 = f16 & 7

    def fire_user(s, buf, sem):
        v = uidx_v[pl.ds(s * RING, L)]
        for j in range(RING):
            i = v[j]
            w = lax.shift_right_logical(i, 7)
            for fb in range(4):
                pltpu.async_copy(
                    ufv_hbm.at[fb, :, pl.ds(w * 128, 128)], buf.at[j, fb], sem)

    def drain_user(buf, sem):
        for j in range(RING):
            pltpu.make_async_copy(
                ufv_hbm.at[:, :, pl.ds(0, 128)], buf.at[j], sem).wait()

    def process_user(k, buf):
        v = uidx_v[pl.ds(k * RING, L)]
        for j in range(RING):
            i = v[j]
            il = jnp.broadcast_to(i & 127, (L,))
            jb = jnp.full((L,), j, jnp.int32)
            u_lo = plsc.load_gather(buf, [jb, band_lo, fr16, il])
            u_hi = plsc.load_gather(buf, [jb, band_hi, fr16, il])
            r = k * RING + j
            urows_v[pl.ds(r * F, L)] = u_lo
            urows_v[pl.ds(r * F + L, L)] = u_hi

    ubufs = (ut0, ut1, ut2)
    usems = (su0, su1, su2)
    fire_user(0, ut0, su0)
    fire_user(1, ut1, su1)

    def user_step(k, carry):
        for ph in range(3):
            @pl.when(k % 3 == ph)
            def _(ph=ph):
                nxt = (ph + 2) % 3

                @pl.when(k + 2 < NSTEP)
                def _():
                    fire_user(k + 2, ubufs[nxt], usems[nxt])
                drain_user(ubufs[ph], usems[ph])
                process_user(k, ubufs[ph])

        return carry

    lax.fori_loop(0, NSTEP, user_step, 0)

    for g in range(NG):
        if g + 1 < NG:
            movie_descs.append(fire_movie(g + 1))
        movie_descs[g].wait()
        mrb = mbufs[g % 2]

        def comp(c, carry, mrb=mrb, g=g):
            sl = pl.ds(g * GR + c * L, L)
            moff = (midx_v[sl] & (MPACK - 1)) * F
            rows = c * L + iota
            ubase = (g * GR + c * L + iota) * F
            acc = jnp.zeros((L,), jnp.float32)
            for f in range(F):
                uv = plsc.load_gather(urows_v, [ubase + f])
                mv = plsc.load_gather(mrb, [rows, moff + f])
                acc = acc + uv * mv
            out_v[sl] = acc
            return carry

        lax.fori_loop(0, GCH, comp, 0)

    pltpu.sync_copy(out_v, out_hbm.at[pl.ds(base, BPW)])


@jax.jit
def kernel(user, movie, user_factors, movie_factors):
    n_users, n_factors = user_factors.shape
    n_movies, _ = movie_factors.shape
    ufv = user_factors.T.reshape(4, 8, n_users)
    mf4 = movie_factors.reshape(n_movies // MPACK, MPACK * n_factors)
    mesh = plsc.VectorSubcoreMesh(
        core_axis_name="c", subcore_axis_name="s",
        num_cores=NC, num_subcores=NS)
    run = pl.kernel(
        _mf_body,
        out_type=jax.ShapeDtypeStruct((BATCH,), jnp.float32),
        mesh=mesh,
        scratch_types=[
            pltpu.VMEM((BPW + L,), jnp.int32),
            pltpu.VMEM((BPW,), jnp.int32),
            pltpu.VMEM((BPW,), jnp.int32),
            pltpu.VMEM((RING, 4, 8, 128), jnp.float32),
            pltpu.VMEM((RING, 4, 8, 128), jnp.float32),
            pltpu.VMEM((RING, 4, 8, 128), jnp.float32),
            pltpu.VMEM((GR, MPACK * F), jnp.float32),
            pltpu.VMEM((GR, MPACK * F), jnp.float32),
            pltpu.VMEM((BPW * F,), jnp.float32),
            pltpu.VMEM((BPW,), jnp.float32),
            pltpu.SemaphoreType.DMA,
            pltpu.SemaphoreType.DMA,
            pltpu.SemaphoreType.DMA,
            pltpu.SemaphoreType.DMA,
            pltpu.SemaphoreType.DMA,
        ],
        compiler_params=pltpu.CompilerParams(needs_layout_passes=False),
    )
    return run(user, movie, ufv, mf4)
